# baseline (speedup 1.0000x reference)
import functools

import jax
import jax.numpy as jnp
from jax import lax
from jax.experimental import pallas as pl
from jax.experimental.pallas import tpu as pltpu
from jax.experimental.pallas import tpu_sc as plsc

_D = 64
_START = 400001
_END = 400002

_NC, _NS = 2, 16
_NW = _NC * _NS
_CHUNK = 128
_LANES = 16
_GPB = 2
_NBUF = 2
_ROWS_B = _GPB * _CHUNK
_DP = 128


def _body(n_groups, idx_hbm, table_hbm, marker_hbm, out_hbm,
          idx_v, marker_v, rows, packs, sem_g, sem_w):
  wid = lax.axis_index("s") * _NC + lax.axis_index("c")
  n_chunks = n_groups * _GPB
  chunk0 = wid * n_chunks

  pltpu.sync_copy(idx_hbm.at[pl.ds(chunk0, n_chunks)], idx_v)
  pltpu.sync_copy(marker_hbm, marker_v)
  m0 = [marker_v[0, pl.ds(k * _LANES, _LANES)] for k in range(_D // _LANES)]
  m1 = [marker_v[1, pl.ds(k * _LANES, _LANES)] for k in range(_D // _LANES)]

  def fire_gather(b, g):
    for u in range(_GPB):
      pltpu.async_copy(table_hbm.at[idx_v.at[g * _GPB + u]],
                       rows[b].at[pl.ds(u * _CHUNK, _CHUNK)], sem_g[b])

  def wait_gather(b, g):
    for u in range(_GPB):
      pltpu.make_async_copy(table_hbm.at[idx_v.at[g * _GPB + u]],
                            rows[b].at[pl.ds(u * _CHUNK, _CHUNK)],
                            sem_g[b]).wait()

  def out_dst(g):
    return out_hbm.at[pl.ds((wid * n_groups + g) * (_ROWS_B // 2),
                            _ROWS_B // 2)]

  def fire_write(b, g):
    pltpu.async_copy(packs[b], out_dst(g), sem_w[b])

  def wait_write(b, g):
    pltpu.make_async_copy(packs[b], out_dst(g), sem_w[b]).wait()

  def pack_group(b):
    h = _D // _LANES
    @pl.loop(0, _ROWS_B // 2 // 8)
    def _pk(pb):
      for r8 in range(8):
        p = pb * 8 + r8
        for k in range(h):
          packs[b][p, pl.ds(k * _LANES, _LANES)] = (
              rows[b][2 * p, pl.ds(k * _LANES, _LANES)])
          packs[b][p, pl.ds(_D + k * _LANES, _LANES)] = (
              rows[b][2 * p + 1, pl.ds(k * _LANES, _LANES)])

  def marker_check(g):
    acc = None
    for u in range(_GPB):
      for q in range(_CHUNK // _LANES):
        vg = idx_v[g * _GPB + u, pl.ds(q * _LANES, _LANES)]
        mg = (vg == _START) | (vg == _END)
        acc = mg if acc is None else (acc | mg)
    return plsc.all_reduce_population_count(acc)[0] > 0

  def fix_markers(b, g, any_hit):
    @pl.when(any_hit)
    def _fix():
      for u in range(_GPB):
        @pl.loop(0, _CHUNK // _LANES)
        def _grp(q):
          vg = idx_v[g * _GPB + u, pl.ds(q * _LANES, _LANES)]
          for r in range(_LANES):
            s = vg[r]
            row = u * _CHUNK + q * _LANES + r

            @pl.when(s == _START)
            def _():
              for k in range(_D // _LANES):
                rows[b][row, pl.ds(k * _LANES, _LANES)] = m0[k]

            @pl.when(s == _END)
            def _():
              for k in range(_D // _LANES):
                rows[b][row, pl.ds(k * _LANES, _LANES)] = m1[k]

  for b in range(_NBUF):
    fire_gather(b, b)

  @pl.loop(0, n_groups // _NBUF)
  def _super(js):
    for bi in range(_NBUF):
      g = js * _NBUF + bi

      bp = (bi - 1) % _NBUF
      can_prefetch = jnp.logical_and(g >= 1, g <= n_groups - _NBUF)

      @pl.when(can_prefetch)
      def _prefetch():
        wait_write(bp, g - 1)
        fire_gather(bp, g + _NBUF - 1)

      any_hit = marker_check(g)
      wait_gather(bi, g)
      fix_markers(bi, g, any_hit)
      pack_group(bi)
      fire_write(bi, g)

  for b in range(_NBUF):
    wait_write(b, n_groups - _NBUF + b)


_VB = 4096


def _pad_body(in_ref, out_ref):
  out_ref[:, 0:_D] = in_ref[...].T


def _transpose_pad(table_t):
  v = table_t.shape[1]
  grid = (v + _VB - 1) // _VB
  return pl.pallas_call(
      _pad_body,
      grid=(grid,),
      in_specs=[pl.BlockSpec((_D, _VB), lambda i: (0, i))],
      out_specs=pl.BlockSpec((_VB, _DP), lambda i: (i, 0)),
      out_shape=jax.ShapeDtypeStruct((v, _DP), jnp.float32),
  )(table_t)


def kernel(idxes, embeddings_weight, marker_weight):
  n_rows = idxes.size
  n_chunks = n_rows // (_NW * _CHUNK)
  n_groups = n_chunks // _GPB
  assert n_rows == _NW * _CHUNK * n_chunks and n_groups % _NBUF == 0
  idx_flat = idxes.reshape(_NW * n_chunks, _CHUNK)
  table_p = _transpose_pad(embeddings_weight.T)
  marker_p = jnp.pad(marker_weight, ((0, 0), (0, _DP - _D)))

  run = pl.kernel(
      functools.partial(_body, n_groups),
      out_type=jax.ShapeDtypeStruct((n_rows // 2, _DP), jnp.float32),
      mesh=plsc.VectorSubcoreMesh(core_axis_name="c", subcore_axis_name="s"),
      compiler_params=pltpu.CompilerParams(
          needs_layout_passes=False, use_tc_tiling_on_sc=True),
      scratch_types=[
          pltpu.VMEM((n_chunks, _CHUNK), jnp.int32),
          pltpu.VMEM((2, _DP), jnp.float32),
          [pltpu.VMEM((_ROWS_B, _DP), jnp.float32) for _ in range(_NBUF)],
          [pltpu.VMEM((_ROWS_B // 2, _DP), jnp.float32) for _ in range(_NBUF)],
          [pltpu.SemaphoreType.DMA for _ in range(_NBUF)],
          [pltpu.SemaphoreType.DMA for _ in range(_NBUF)],
      ],
  )
  out = run(idx_flat, table_p, marker_p)
  return out.reshape(idxes.shape + (_D,))


# speedup vs baseline: 1.3733x; 1.3733x over previous
import functools

import jax
import jax.numpy as jnp
from jax import lax
from jax.experimental import pallas as pl
from jax.experimental.pallas import tpu as pltpu
from jax.experimental.pallas import tpu_sc as plsc

_D = 64
_START = 400001
_END = 400002

_NC, _NS = 2, 16
_NW = _NC * _NS
_CHUNK = 128
_LANES = 16
_GPB = 2
_NBUF = 2
_ROWS_B = _GPB * _CHUNK
_DP = 128


def _body(n_groups, idx_hbm, table_hbm, marker_hbm, out_hbm,
          idx_v, marker_v, rows, sem_g, sem_w):
  wid = lax.axis_index("s") * _NC + lax.axis_index("c")
  n_chunks = n_groups * _GPB

  pltpu.sync_copy(idx_hbm.at[wid], idx_v)
  pltpu.sync_copy(marker_hbm, marker_v)
  m0 = [marker_v[0, pl.ds(k * _LANES, _LANES)] for k in range(_D // _LANES)]
  m1 = [marker_v[1, pl.ds(k * _LANES, _LANES)] for k in range(_D // _LANES)]

  def fire_gather(b, g):
    for u in range(_GPB):
      pltpu.async_copy(table_hbm.at[idx_v.at[g * _GPB + u]],
                       rows[b].at[pl.ds(u * _CHUNK, _CHUNK)], sem_g[b])

  def wait_gather(b, g):
    for u in range(_GPB):
      pltpu.make_async_copy(table_hbm.at[idx_v.at[g * _GPB + u]],
                            rows[b].at[pl.ds(u * _CHUNK, _CHUNK)],
                            sem_g[b]).wait()

  def out_dst(g):
    return out_hbm.at[pl.ds((wid * n_groups + g) * _ROWS_B, _ROWS_B)]

  def fire_write(b, g):
    pltpu.async_copy(rows[b], out_dst(g), sem_w[b])

  def wait_write(b, g):
    pltpu.make_async_copy(rows[b], out_dst(g), sem_w[b]).wait()

  def marker_check(g):
    acc = None
    for u in range(_GPB):
      for q in range(_CHUNK // _LANES):
        vg = idx_v[g * _GPB + u, pl.ds(q * _LANES, _LANES)]
        mg = (vg == _START) | (vg == _END)
        acc = mg if acc is None else (acc | mg)
    return plsc.all_reduce_population_count(acc)[0] > 0

  def fix_markers(b, g, any_hit):
    @pl.when(any_hit)
    def _fix():
      for u in range(_GPB):
        @pl.loop(0, _CHUNK // _LANES)
        def _grp(q):
          vg = idx_v[g * _GPB + u, pl.ds(q * _LANES, _LANES)]
          for r in range(_LANES):
            s = vg[r]
            row = u * _CHUNK + q * _LANES + r

            @pl.when(s == _START)
            def _():
              for k in range(_D // _LANES):
                rows[b][row, pl.ds(k * _LANES, _LANES)] = m0[k]

            @pl.when(s == _END)
            def _():
              for k in range(_D // _LANES):
                rows[b][row, pl.ds(k * _LANES, _LANES)] = m1[k]

  for b in range(_NBUF):
    fire_gather(b, b)

  @pl.loop(0, n_groups // _NBUF)
  def _super(js):
    for bi in range(_NBUF):
      g = js * _NBUF + bi

      bp = (bi - 1) % _NBUF
      can_prefetch = jnp.logical_and(g >= 1, g <= n_groups - _NBUF)

      @pl.when(can_prefetch)
      def _prefetch():
        wait_write(bp, g - 1)
        fire_gather(bp, g + _NBUF - 1)

      any_hit = marker_check(g)
      wait_gather(bi, g)
      fix_markers(bi, g, any_hit)
      fire_write(bi, g)

  for b in range(_NBUF):
    wait_write(b, n_groups - _NBUF + b)


_VB = 4096


def _pad_body(in_ref, out_ref):
  out_ref[:, 0:_D] = in_ref[...].T


def _transpose_pad(table_t):
  v = table_t.shape[1]
  grid = (v + _VB - 1) // _VB
  return pl.pallas_call(
      _pad_body,
      grid=(grid,),
      in_specs=[pl.BlockSpec((_D, _VB), lambda i: (0, i))],
      out_specs=pl.BlockSpec((_VB, _DP), lambda i: (i, 0)),
      out_shape=jax.ShapeDtypeStruct((v, _DP), jnp.float32),
  )(table_t)


_SPLIT = 2


def kernel(idxes, embeddings_weight, marker_weight):
  n_batch, n_seq = idxes.shape
  n_rows = idxes.size
  n_chunks = n_rows // (_NW * _CHUNK * _SPLIT)
  n_groups = n_chunks // _GPB
  assert n_rows == _SPLIT * _NW * _CHUNK * n_chunks and n_groups % _NBUF == 0
  assert n_batch % _SPLIT == 0 and (n_batch // _SPLIT) * n_seq % _CHUNK == 0
  idx_flat = idxes.reshape(_SPLIT, _NW, n_chunks, _CHUNK)
  table_p = _transpose_pad(embeddings_weight.T)
  marker_p = jnp.pad(marker_weight, ((0, 0), (0, _DP - _D)))

  run = pl.kernel(
      functools.partial(_body, n_groups),
      out_type=jax.ShapeDtypeStruct((n_rows // _SPLIT, _DP), jnp.float32),
      mesh=plsc.VectorSubcoreMesh(core_axis_name="c", subcore_axis_name="s"),
      compiler_params=pltpu.CompilerParams(
          needs_layout_passes=False, use_tc_tiling_on_sc=True),
      scratch_types=[
          pltpu.VMEM((n_chunks, _CHUNK), jnp.int32),
          pltpu.VMEM((2, _DP), jnp.float32),
          [pltpu.VMEM((_ROWS_B, _DP), jnp.float32) for _ in range(_NBUF)],
          [pltpu.SemaphoreType.DMA for _ in range(_NBUF)],
          [pltpu.SemaphoreType.DMA for _ in range(_NBUF)],
      ],
  )
  bs = n_batch // _SPLIT
  parts = [
      run(idx_flat[k], table_p, marker_p)[:, :_D].reshape(bs, n_seq, _D)
      for k in range(_SPLIT)
  ]
  return jnp.concatenate(parts, axis=0)


# speedup vs baseline: 1.7000x; 1.2379x over previous
import functools

import jax
import jax.numpy as jnp
from jax import lax
from jax.experimental import pallas as pl
from jax.experimental.pallas import tpu as pltpu
from jax.experimental.pallas import tpu_sc as plsc

_D = 64
_START = 400001
_END = 400002

_NC, _NS = 2, 16
_NW = _NC * _NS
_CHUNK = 128
_LANES = 16
_GPB = 2
_NBUF = 2
_ROWS_B = _GPB * _CHUNK
_DP = 128


def _body(n_groups, idx_hbm, table_hbm, marker_hbm, out_hbm,
          idx_v, marker_v, rows, sem_g, sem_w):
  wid = lax.axis_index("s") * _NC + lax.axis_index("c")
  n_chunks = n_groups * _GPB
  chunk0 = wid * n_chunks

  pltpu.sync_copy(idx_hbm.at[pl.ds(chunk0, n_chunks)], idx_v)
  pltpu.sync_copy(marker_hbm, marker_v)
  m0 = [marker_v[0, pl.ds(k * _LANES, _LANES)] for k in range(_D // _LANES)]
  m1 = [marker_v[1, pl.ds(k * _LANES, _LANES)] for k in range(_D // _LANES)]

  def fire_gather(b, g):
    for u in range(_GPB):
      pltpu.async_copy(table_hbm.at[idx_v.at[g * _GPB + u]],
                       rows[b].at[pl.ds(u * _CHUNK, _CHUNK)], sem_g[b])

  def wait_gather(b, g):
    for u in range(_GPB):
      pltpu.make_async_copy(table_hbm.at[idx_v.at[g * _GPB + u]],
                            rows[b].at[pl.ds(u * _CHUNK, _CHUNK)],
                            sem_g[b]).wait()

  def out_dst(g):
    return out_hbm.at[pl.ds((wid * n_groups + g) * _ROWS_B, _ROWS_B)]

  def fire_write(b, g):
    pltpu.async_copy(rows[b], out_dst(g), sem_w[b])

  def wait_write(b, g):
    pltpu.make_async_copy(rows[b], out_dst(g), sem_w[b]).wait()

  def marker_check(g):
    acc = None
    for u in range(_GPB):
      for q in range(_CHUNK // _LANES):
        vg = idx_v[g * _GPB + u, pl.ds(q * _LANES, _LANES)]
        mg = (vg == _START) | (vg == _END)
        acc = mg if acc is None else (acc | mg)
    return plsc.all_reduce_population_count(acc)[0] > 0

  def fix_markers(b, g, any_hit):
    @pl.when(any_hit)
    def _fix():
      for u in range(_GPB):
        @pl.loop(0, _CHUNK // _LANES)
        def _grp(q):
          vg = idx_v[g * _GPB + u, pl.ds(q * _LANES, _LANES)]
          for r in range(_LANES):
            s = vg[r]
            row = u * _CHUNK + q * _LANES + r

            @pl.when(s == _START)
            def _():
              for k in range(_D // _LANES):
                rows[b][row, pl.ds(k * _LANES, _LANES)] = m0[k]

            @pl.when(s == _END)
            def _():
              for k in range(_D // _LANES):
                rows[b][row, pl.ds(k * _LANES, _LANES)] = m1[k]

  for b in range(_NBUF):
    fire_gather(b, b)

  @pl.loop(0, n_groups // _NBUF)
  def _super(js):
    for bi in range(_NBUF):
      g = js * _NBUF + bi

      bp = (bi - 1) % _NBUF
      can_prefetch = jnp.logical_and(g >= 1, g <= n_groups - _NBUF)

      @pl.when(can_prefetch)
      def _prefetch():
        wait_write(bp, g - 1)
        fire_gather(bp, g + _NBUF - 1)

      any_hit = marker_check(g)
      wait_gather(bi, g)
      fix_markers(bi, g, any_hit)
      fire_write(bi, g)

  for b in range(_NBUF):
    wait_write(b, n_groups - _NBUF + b)


_VB = 4096


def _pad_body(in_ref, out_ref):
  out_ref[:, 0:_D] = in_ref[...].T


def _transpose_pad(table_t):
  v = table_t.shape[1]
  grid = (v + _VB - 1) // _VB
  return pl.pallas_call(
      _pad_body,
      grid=(grid,),
      in_specs=[pl.BlockSpec((_D, _VB), lambda i: (0, i))],
      out_specs=pl.BlockSpec((_VB, _DP), lambda i: (i, 0)),
      out_shape=jax.ShapeDtypeStruct((v, _DP), jnp.float32),
  )(table_t)


def kernel(idxes, embeddings_weight, marker_weight):
  n_rows = idxes.size
  n_chunks = n_rows // (_NW * _CHUNK)
  n_groups = n_chunks // _GPB
  assert n_rows == _NW * _CHUNK * n_chunks and n_groups % _NBUF == 0
  idx_flat = idxes.reshape(_NW * n_chunks, _CHUNK)
  table_p = _transpose_pad(embeddings_weight.T)
  marker_p = jnp.pad(marker_weight, ((0, 0), (0, _DP - _D)))

  run = pl.kernel(
      functools.partial(_body, n_groups),
      out_type=jax.ShapeDtypeStruct((n_rows, _DP), jnp.float32),
      mesh=plsc.VectorSubcoreMesh(core_axis_name="c", subcore_axis_name="s"),
      compiler_params=pltpu.CompilerParams(
          needs_layout_passes=False, use_tc_tiling_on_sc=True),
      scratch_types=[
          pltpu.VMEM((n_chunks, _CHUNK), jnp.int32),
          pltpu.VMEM((2, _DP), jnp.float32),
          [pltpu.VMEM((_ROWS_B, _DP), jnp.float32) for _ in range(_NBUF)],
          [pltpu.SemaphoreType.DMA for _ in range(_NBUF)],
          [pltpu.SemaphoreType.DMA for _ in range(_NBUF)],
      ],
  )
  out = run(idx_flat, table_p, marker_p)
  return out[:, :_D].reshape(idxes.shape + (_D,))


# speedup vs baseline: 1.7732x; 1.0431x over previous
import functools

import jax
import jax.numpy as jnp
from jax import lax
from jax.experimental import pallas as pl
from jax.experimental.pallas import tpu as pltpu
from jax.experimental.pallas import tpu_sc as plsc

_D = 64
_START = 400001
_END = 400002

_NC, _NS = 2, 16
_NW = _NC * _NS
_CHUNK = 128
_LANES = 16
_GPB = 2
_NBUF = 2
_ROWS_B = _GPB * _CHUNK
_DP = 128


def _body(n_groups, idx_hbm, table_hbm, marker_hbm, out_hbm,
          idx_v, marker_v, rows, sem_g, sem_w):
  wid = lax.axis_index("s") * _NC + lax.axis_index("c")
  n_chunks = n_groups * _GPB
  chunk0 = wid * n_chunks

  pltpu.sync_copy(idx_hbm.at[pl.ds(chunk0, n_chunks)], idx_v)
  pltpu.sync_copy(marker_hbm, marker_v)
  m0 = [marker_v[0, pl.ds(k * _LANES, _LANES)] for k in range(_D // _LANES)]
  m1 = [marker_v[1, pl.ds(k * _LANES, _LANES)] for k in range(_D // _LANES)]

  def fire_gather(b, g):
    for u in range(_GPB):
      pltpu.async_copy(table_hbm.at[idx_v.at[g * _GPB + u]],
                       rows[b].at[pl.ds(u * _CHUNK, _CHUNK)], sem_g[b])

  def wait_gather(b, g):
    for u in range(_GPB):
      pltpu.make_async_copy(table_hbm.at[idx_v.at[g * _GPB + u]],
                            rows[b].at[pl.ds(u * _CHUNK, _CHUNK)],
                            sem_g[b]).wait()

  def out_dst(g):
    return out_hbm.at[pl.ds((wid * n_groups + g) * _ROWS_B, _ROWS_B)]

  def fire_write(b, g):
    pltpu.async_copy(rows[b], out_dst(g), sem_w[b])

  def wait_write(b, g):
    pltpu.make_async_copy(rows[b], out_dst(g), sem_w[b]).wait()

  def marker_check(g):
    acc = None
    for u in range(_GPB):
      for q in range(_CHUNK // _LANES):
        vg = idx_v[g * _GPB + u, pl.ds(q * _LANES, _LANES)]
        mg = (vg == _START) | (vg == _END)
        acc = mg if acc is None else (acc | mg)
    return plsc.all_reduce_population_count(acc)[0] > 0

  def fix_markers(b, g, any_hit):
    @pl.when(any_hit)
    def _fix():
      for u in range(_GPB):
        @pl.loop(0, _CHUNK // _LANES)
        def _grp(q):
          vg = idx_v[g * _GPB + u, pl.ds(q * _LANES, _LANES)]
          for r in range(_LANES):
            s = vg[r]
            row = u * _CHUNK + q * _LANES + r

            @pl.when(s == _START)
            def _():
              for k in range(_D // _LANES):
                rows[b][row, pl.ds(k * _LANES, _LANES)] = m0[k]

            @pl.when(s == _END)
            def _():
              for k in range(_D // _LANES):
                rows[b][row, pl.ds(k * _LANES, _LANES)] = m1[k]

  for b in range(_NBUF):
    fire_gather(b, b)

  @pl.loop(0, n_groups // _NBUF)
  def _super(js):
    for bi in range(_NBUF):
      g = js * _NBUF + bi

      bp = (bi - 1) % _NBUF
      can_prefetch = jnp.logical_and(g >= 1, g <= n_groups - _NBUF)

      @pl.when(can_prefetch)
      def _prefetch():
        wait_write(bp, g - 1)
        fire_gather(bp, g + _NBUF - 1)

      any_hit = marker_check(g)
      wait_gather(bi, g)
      fix_markers(bi, g, any_hit)
      fire_write(bi, g)

  for b in range(_NBUF):
    wait_write(b, n_groups - _NBUF + b)


_VB = 8192


def _pad_body(in_ref, out_ref):
  out_ref[:, 0:_D] = in_ref[...].T


def _transpose_pad(table_t):
  v = table_t.shape[1]
  grid = (v + _VB - 1) // _VB
  return pl.pallas_call(
      _pad_body,
      grid=(grid,),
      in_specs=[pl.BlockSpec((_D, _VB), lambda i: (0, i))],
      out_specs=pl.BlockSpec((_VB, _DP), lambda i: (i, 0)),
      out_shape=jax.ShapeDtypeStruct((v, _DP), jnp.float32),
  )(table_t)


def kernel(idxes, embeddings_weight, marker_weight):
  n_rows = idxes.size
  n_chunks = n_rows // (_NW * _CHUNK)
  n_groups = n_chunks // _GPB
  assert n_rows == _NW * _CHUNK * n_chunks and n_groups % _NBUF == 0
  idx_flat = idxes.reshape(_NW * n_chunks, _CHUNK)
  table_p = _transpose_pad(embeddings_weight.T)
  marker_p = jnp.pad(marker_weight, ((0, 0), (0, _DP - _D)))

  run = pl.kernel(
      functools.partial(_body, n_groups),
      out_type=jax.ShapeDtypeStruct((n_rows, _DP), jnp.float32),
      mesh=plsc.VectorSubcoreMesh(core_axis_name="c", subcore_axis_name="s"),
      compiler_params=pltpu.CompilerParams(
          needs_layout_passes=False, use_tc_tiling_on_sc=True),
      scratch_types=[
          pltpu.VMEM((n_chunks, _CHUNK), jnp.int32),
          pltpu.VMEM((2, _DP), jnp.float32),
          [pltpu.VMEM((_ROWS_B, _DP), jnp.float32) for _ in range(_NBUF)],
          [pltpu.SemaphoreType.DMA for _ in range(_NBUF)],
          [pltpu.SemaphoreType.DMA for _ in range(_NBUF)],
      ],
  )
  out = run(idx_flat, table_p, marker_p)
  return out[:, :_D].reshape(idxes.shape + (_D,))


# speedup vs baseline: 1.8094x; 1.0204x over previous
import functools

import jax
import jax.numpy as jnp
from jax import lax
from jax.experimental import pallas as pl
from jax.experimental.pallas import tpu as pltpu
from jax.experimental.pallas import tpu_sc as plsc

_D = 64
_START = 400001
_END = 400002

_NC, _NS = 2, 16
_NW = _NC * _NS
_CHUNK = 128
_LANES = 16
_GPB = 2
_NBUF = 2
_ROWS_B = _GPB * _CHUNK
_DP = 128


def _body(n_groups, idx_hbm, table_hbm, marker_hbm, out_hbm,
          idx_v, marker_v, rows, sem_g, sem_w):
  wid = lax.axis_index("s") * _NC + lax.axis_index("c")
  n_chunks = n_groups * _GPB
  chunk0 = wid * n_chunks

  pltpu.sync_copy(idx_hbm.at[pl.ds(chunk0, n_chunks)], idx_v)
  pltpu.sync_copy(marker_hbm, marker_v)
  m0 = [marker_v[0, pl.ds(k * _LANES, _LANES)] for k in range(_D // _LANES)]
  m1 = [marker_v[1, pl.ds(k * _LANES, _LANES)] for k in range(_D // _LANES)]

  def fire_gather(b, g):
    for u in range(_GPB):
      pltpu.async_copy(table_hbm.at[idx_v.at[g * _GPB + u]],
                       rows[b].at[pl.ds(u * _CHUNK, _CHUNK)], sem_g[b])

  def wait_gather(b, g):
    for u in range(_GPB):
      pltpu.make_async_copy(table_hbm.at[idx_v.at[g * _GPB + u]],
                            rows[b].at[pl.ds(u * _CHUNK, _CHUNK)],
                            sem_g[b]).wait()

  def out_dst(g):
    return out_hbm.at[pl.ds((wid * n_groups + g) * _ROWS_B, _ROWS_B)]

  def fire_write(b, g):
    pltpu.async_copy(rows[b], out_dst(g), sem_w[b])

  def wait_write(b, g):
    pltpu.make_async_copy(rows[b], out_dst(g), sem_w[b]).wait()

  def marker_check(g):
    acc = None
    for u in range(_GPB):
      for q in range(_CHUNK // _LANES):
        vg = idx_v[g * _GPB + u, pl.ds(q * _LANES, _LANES)]
        mg = (vg == _START) | (vg == _END)
        acc = mg if acc is None else (acc | mg)
    return plsc.all_reduce_population_count(acc)[0] > 0

  def fix_markers(b, g, any_hit):
    @pl.when(any_hit)
    def _fix():
      for u in range(_GPB):
        @pl.loop(0, _CHUNK // _LANES)
        def _grp(q):
          vg = idx_v[g * _GPB + u, pl.ds(q * _LANES, _LANES)]
          for r in range(_LANES):
            s = vg[r]
            row = u * _CHUNK + q * _LANES + r

            @pl.when(s == _START)
            def _():
              for k in range(_D // _LANES):
                rows[b][row, pl.ds(k * _LANES, _LANES)] = m0[k]

            @pl.when(s == _END)
            def _():
              for k in range(_D // _LANES):
                rows[b][row, pl.ds(k * _LANES, _LANES)] = m1[k]

  for b in range(_NBUF):
    fire_gather(b, b)

  @pl.loop(0, n_groups // _NBUF)
  def _super(js):
    for bi in range(_NBUF):
      g = js * _NBUF + bi

      bp = (bi - 1) % _NBUF
      can_prefetch = jnp.logical_and(g >= 1, g <= n_groups - _NBUF)

      @pl.when(can_prefetch)
      def _prefetch():
        wait_write(bp, g - 1)
        fire_gather(bp, g + _NBUF - 1)

      any_hit = marker_check(g)
      wait_gather(bi, g)
      fix_markers(bi, g, any_hit)
      fire_write(bi, g)

  for b in range(_NBUF):
    wait_write(b, n_groups - _NBUF + b)


_VB = 16384


def _pad_body(in_ref, out_ref):
  out_ref[:, 0:_D] = in_ref[...].T


def _transpose_pad(table_t):
  v = table_t.shape[1]
  grid = (v + _VB - 1) // _VB
  return pl.pallas_call(
      _pad_body,
      grid=(grid,),
      in_specs=[pl.BlockSpec((_D, _VB), lambda i: (0, i))],
      out_specs=pl.BlockSpec((_VB, _DP), lambda i: (i, 0)),
      out_shape=jax.ShapeDtypeStruct((v, _DP), jnp.float32),
  )(table_t)


def kernel(idxes, embeddings_weight, marker_weight):
  n_rows = idxes.size
  n_chunks = n_rows // (_NW * _CHUNK)
  n_groups = n_chunks // _GPB
  assert n_rows == _NW * _CHUNK * n_chunks and n_groups % _NBUF == 0
  idx_flat = idxes.reshape(_NW * n_chunks, _CHUNK)
  table_p = _transpose_pad(embeddings_weight.T)
  marker_p = jnp.pad(marker_weight, ((0, 0), (0, _DP - _D)))

  run = pl.kernel(
      functools.partial(_body, n_groups),
      out_type=jax.ShapeDtypeStruct((n_rows, _DP), jnp.float32),
      mesh=plsc.VectorSubcoreMesh(core_axis_name="c", subcore_axis_name="s"),
      compiler_params=pltpu.CompilerParams(
          needs_layout_passes=False, use_tc_tiling_on_sc=True),
      scratch_types=[
          pltpu.VMEM((n_chunks, _CHUNK), jnp.int32),
          pltpu.VMEM((2, _DP), jnp.float32),
          [pltpu.VMEM((_ROWS_B, _DP), jnp.float32) for _ in range(_NBUF)],
          [pltpu.SemaphoreType.DMA for _ in range(_NBUF)],
          [pltpu.SemaphoreType.DMA for _ in range(_NBUF)],
      ],
  )
  out = run(idx_flat, table_p, marker_p)
  return out[:, :_D].reshape(idxes.shape + (_D,))


# speedup vs baseline: 1.8175x; 1.0045x over previous
import functools

import jax
import jax.numpy as jnp
from jax import lax
from jax.experimental import pallas as pl
from jax.experimental.pallas import tpu as pltpu
from jax.experimental.pallas import tpu_sc as plsc

_D = 64
_START = 400001
_END = 400002

_NC, _NS = 2, 16
_NW = _NC * _NS
_CHUNK = 128
_LANES = 16
_GPB = 2
_NBUF = 2
_ROWS_B = _GPB * _CHUNK
_DP = 128


def _body(n_groups, idx_hbm, table_hbm, marker_hbm, out_hbm,
          idx_v, marker_v, rows, sem_g, sem_w):
  wid = lax.axis_index("s") * _NC + lax.axis_index("c")
  n_chunks = n_groups * _GPB
  chunk0 = wid * n_chunks

  pltpu.sync_copy(idx_hbm.at[pl.ds(chunk0, n_chunks)], idx_v)
  pltpu.sync_copy(marker_hbm, marker_v)
  m0 = [marker_v[0, pl.ds(k * _LANES, _LANES)] for k in range(_D // _LANES)]
  m1 = [marker_v[1, pl.ds(k * _LANES, _LANES)] for k in range(_D // _LANES)]

  def fire_gather(b, g):
    for u in range(_GPB):
      pltpu.async_copy(table_hbm.at[idx_v.at[g * _GPB + u]],
                       rows[b].at[pl.ds(u * _CHUNK, _CHUNK)], sem_g[b])

  def wait_gather(b, g):
    for u in range(_GPB):
      pltpu.make_async_copy(table_hbm.at[idx_v.at[g * _GPB + u]],
                            rows[b].at[pl.ds(u * _CHUNK, _CHUNK)],
                            sem_g[b]).wait()

  def out_dst(g):
    return out_hbm.at[pl.ds((wid * n_groups + g) * _ROWS_B, _ROWS_B)]

  def fire_write(b, g):
    pltpu.async_copy(rows[b], out_dst(g), sem_w[b])

  def wait_write(b, g):
    pltpu.make_async_copy(rows[b], out_dst(g), sem_w[b]).wait()

  def marker_check(g):
    acc = None
    for u in range(_GPB):
      for q in range(_CHUNK // _LANES):
        vg = idx_v[g * _GPB + u, pl.ds(q * _LANES, _LANES)]
        mg = (vg == _START) | (vg == _END)
        acc = mg if acc is None else (acc | mg)
    return plsc.all_reduce_population_count(acc)[0] > 0

  def fix_markers(b, g, any_hit):
    @pl.when(any_hit)
    def _fix():
      for u in range(_GPB):
        @pl.loop(0, _CHUNK // _LANES)
        def _grp(q):
          vg = idx_v[g * _GPB + u, pl.ds(q * _LANES, _LANES)]
          for r in range(_LANES):
            s = vg[r]
            row = u * _CHUNK + q * _LANES + r

            @pl.when(s == _START)
            def _():
              for k in range(_D // _LANES):
                rows[b][row, pl.ds(k * _LANES, _LANES)] = m0[k]

            @pl.when(s == _END)
            def _():
              for k in range(_D // _LANES):
                rows[b][row, pl.ds(k * _LANES, _LANES)] = m1[k]

  for b in range(_NBUF):
    fire_gather(b, b)

  @pl.loop(0, n_groups // _NBUF)
  def _super(js):
    for bi in range(_NBUF):
      g = js * _NBUF + bi

      bp = (bi - 1) % _NBUF
      can_prefetch = jnp.logical_and(g >= 1, g <= n_groups - _NBUF)

      @pl.when(can_prefetch)
      def _prefetch():
        wait_write(bp, g - 1)
        fire_gather(bp, g + _NBUF - 1)

      any_hit = marker_check(g)
      wait_gather(bi, g)
      fix_markers(bi, g, any_hit)
      fire_write(bi, g)

  for b in range(_NBUF):
    wait_write(b, n_groups - _NBUF + b)


_VB = 32768


def _pad_body(in_ref, out_ref):
  out_ref[:, 0:_D] = in_ref[...].T


def _transpose_pad(table_t):
  v = table_t.shape[1]
  grid = (v + _VB - 1) // _VB
  return pl.pallas_call(
      _pad_body,
      grid=(grid,),
      in_specs=[pl.BlockSpec((_D, _VB), lambda i: (0, i))],
      out_specs=pl.BlockSpec((_VB, _DP), lambda i: (i, 0)),
      out_shape=jax.ShapeDtypeStruct((v, _DP), jnp.float32),
  )(table_t)


def kernel(idxes, embeddings_weight, marker_weight):
  n_rows = idxes.size
  n_chunks = n_rows // (_NW * _CHUNK)
  n_groups = n_chunks // _GPB
  assert n_rows == _NW * _CHUNK * n_chunks and n_groups % _NBUF == 0
  idx_flat = idxes.reshape(_NW * n_chunks, _CHUNK)
  table_p = _transpose_pad(embeddings_weight.T)
  marker_p = jnp.pad(marker_weight, ((0, 0), (0, _DP - _D)))

  run = pl.kernel(
      functools.partial(_body, n_groups),
      out_type=jax.ShapeDtypeStruct((n_rows, _DP), jnp.float32),
      mesh=plsc.VectorSubcoreMesh(core_axis_name="c", subcore_axis_name="s"),
      compiler_params=pltpu.CompilerParams(
          needs_layout_passes=False, use_tc_tiling_on_sc=True),
      scratch_types=[
          pltpu.VMEM((n_chunks, _CHUNK), jnp.int32),
          pltpu.VMEM((2, _DP), jnp.float32),
          [pltpu.VMEM((_ROWS_B, _DP), jnp.float32) for _ in range(_NBUF)],
          [pltpu.SemaphoreType.DMA for _ in range(_NBUF)],
          [pltpu.SemaphoreType.DMA for _ in range(_NBUF)],
      ],
  )
  out = run(idx_flat, table_p, marker_p)
  return out[:, :_D].reshape(idxes.shape + (_D,))
